# Initial kernel scaffold; baseline (speedup 1.0000x reference)
import jax
import jax.numpy as jnp
from jax.experimental import pallas as pl


def kernel(input, weight):
    raise NotImplementedError("write your pallas kernel here")


# speedup vs baseline: 248.1780x; 248.1780x over previous
import functools

import jax
import jax.numpy as jnp
from jax import lax
from jax.experimental import pallas as pl
from jax.experimental.pallas import tpu as pltpu
from jax.experimental.pallas import tpu_sc as plsc

N = 3276800
NC = 2
NS = 16
L = 16
NW = NC * NS
PER_W = N // NW
VECS = PER_W // L

_mesh = plsc.VectorSubcoreMesh(core_axis_name="c", subcore_axis_name="s")


@functools.partial(
    pl.kernel,
    mesh=_mesh,
    out_type=jax.ShapeDtypeStruct((NW, L), jnp.int32),
    scratch_types=[
        pltpu.VMEM((PER_W,), jnp.int32),
        pltpu.VMEM((L,), jnp.int32),
    ],
)
def _sc_count(in_hbm, out_hbm, buf_v, part_v):
    wid = lax.axis_index("s") * NC + lax.axis_index("c")
    base = wid * PER_W
    pltpu.sync_copy(in_hbm.at[pl.ds(base, PER_W)], buf_v)

    def body(i, acc):
        return acc + buf_v[pl.ds(i * L, L)]

    acc = lax.fori_loop(0, VECS, body, jnp.zeros((L,), jnp.int32))
    part_v[...] = acc
    pltpu.sync_copy(part_v, out_hbm.at[wid])


def _combine_body(part_ref, w_ref, o_ref):
    s = jnp.sum(part_ref[...]).astype(jnp.float32)
    frac = s * (1.0 / N)
    w = w_ref[...]
    o_ref[...] = (1.0 - frac) * w[0:1, :] + frac * w[1:2, :]


def _combine(partials, weight):
    return pl.pallas_call(
        _combine_body,
        out_shape=jax.ShapeDtypeStruct((1, 3), jnp.float32),
    )(partials, weight)


def kernel(input, weight):
    idx = input if input.dtype == jnp.int32 else input.astype(jnp.int32)
    partials = _sc_count(idx)
    return _combine(partials, weight)


# speedup vs baseline: 437.7997x; 1.7641x over previous
import functools

import jax
import jax.numpy as jnp
from jax import lax
from jax.experimental import pallas as pl
from jax.experimental.pallas import tpu as pltpu
from jax.experimental.pallas import tpu_sc as plsc

N = 3276800
NC = 2
NS = 16
L = 16
NW = NC * NS
PER_W = N // NW
VECS = PER_W // L

_mesh = plsc.VectorSubcoreMesh(core_axis_name="c", subcore_axis_name="s")

CHUNK = 12800
NCH = PER_W // CHUNK
U = 8


@functools.partial(
    pl.kernel,
    mesh=_mesh,
    out_type=jax.ShapeDtypeStruct((NW, L), jnp.int32),
    scratch_types=[
        pltpu.VMEM((CHUNK,), jnp.int32),
        pltpu.VMEM((CHUNK,), jnp.int32),
        pltpu.VMEM((L,), jnp.int32),
        pltpu.SemaphoreType.DMA,
        pltpu.SemaphoreType.DMA,
    ],
)
def _sc_count(in_hbm, out_hbm, buf0, buf1, part_v, sem0, sem1):
    wid = lax.axis_index("s") * NC + lax.axis_index("c")
    base = wid * PER_W
    bufs, sems = (buf0, buf1), (sem0, sem1)
    handles = [
        pltpu.async_copy(in_hbm.at[pl.ds(base, CHUNK)], buf0, sem0),
        pltpu.async_copy(in_hbm.at[pl.ds(base + CHUNK, CHUNK)], buf1, sem1),
    ]
    accs = tuple(jnp.zeros((L,), jnp.int32) for _ in range(U))
    for g in range(NCH):
        b = g % 2
        handles[b].wait()
        buf = bufs[b]

        def body(i, accs_t, buf=buf):
            off = i * (U * L)
            return tuple(accs_t[u] + buf[pl.ds(off + u * L, L)] for u in range(U))

        accs = lax.fori_loop(0, CHUNK // (U * L), body, accs)
        if g + 2 < NCH:
            handles[b] = pltpu.async_copy(
                in_hbm.at[pl.ds(base + (g + 2) * CHUNK, CHUNK)], bufs[b], sems[b]
            )
    acc = accs[0]
    for u in range(1, U):
        acc = acc + accs[u]
    part_v[...] = acc
    pltpu.sync_copy(part_v, out_hbm.at[wid])


def _combine_body(part_ref, w_ref, o_ref):
    s = jnp.sum(part_ref[...]).astype(jnp.float32)
    frac = s * (1.0 / N)
    w = w_ref[...]
    o_ref[...] = (1.0 - frac) * w[0:1, :] + frac * w[1:2, :]


def _combine(partials, weight):
    return pl.pallas_call(
        _combine_body,
        out_shape=jax.ShapeDtypeStruct((1, 3), jnp.float32),
    )(partials, weight)


def kernel(input, weight):
    idx = input if input.dtype == jnp.int32 else input.astype(jnp.int32)
    partials = _sc_count(idx)
    return _combine(partials, weight)


# speedup vs baseline: 457.3198x; 1.0446x over previous
import functools

import jax
import jax.numpy as jnp
from jax import lax
from jax.experimental import pallas as pl
from jax.experimental.pallas import tpu as pltpu
from jax.experimental.pallas import tpu_sc as plsc

N = 3276800
NC = 2
NS = 16
L = 16
NW = NC * NS
PER_W = N // NW
VECS = PER_W // L

_mesh = plsc.VectorSubcoreMesh(core_axis_name="c", subcore_axis_name="s")

CHUNK = 12800
NCH = PER_W // CHUNK
U = 8


@functools.partial(
    pl.kernel,
    mesh=_mesh,
    out_type=jax.ShapeDtypeStruct((NW, L), jnp.int32),
    scratch_types=[
        pltpu.VMEM((PER_W,), jnp.int32),
        pltpu.VMEM((L,), jnp.int32),
    ] + [pltpu.SemaphoreType.DMA] * NCH,
)
def _sc_count(in_hbm, out_hbm, buf_v, part_v, *sems):
    wid = lax.axis_index("s") * NC + lax.axis_index("c")
    base = wid * PER_W
    handles = [
        pltpu.async_copy(
            in_hbm.at[pl.ds(base + g * CHUNK, CHUNK)],
            buf_v.at[pl.ds(g * CHUNK, CHUNK)],
            sems[g],
        )
        for g in range(NCH)
    ]
    accs = tuple(jnp.zeros((L,), jnp.int32) for _ in range(U))
    for g in range(NCH):
        handles[g].wait()

        def body(i, accs_t, goff=g * CHUNK):
            off = goff + i * (U * L)
            return tuple(accs_t[u] + buf_v[pl.ds(off + u * L, L)] for u in range(U))

        accs = lax.fori_loop(0, CHUNK // (U * L), body, accs)
    acc = accs[0]
    for u in range(1, U):
        acc = acc + accs[u]
    part_v[...] = acc
    pltpu.sync_copy(part_v, out_hbm.at[wid])


def _combine_body(part_ref, w_ref, o_ref):
    s = jnp.sum(part_ref[...]).astype(jnp.float32)
    frac = s * (1.0 / N)
    w = w_ref[...]
    o_ref[...] = (1.0 - frac) * w[0:1, :] + frac * w[1:2, :]


def _combine(partials, weight):
    return pl.pallas_call(
        _combine_body,
        out_shape=jax.ShapeDtypeStruct((1, 3), jnp.float32),
    )(partials, weight)


def kernel(input, weight):
    idx = input if input.dtype == jnp.int32 else input.astype(jnp.int32)
    partials = _sc_count(idx)
    return _combine(partials, weight)
